# Initial kernel scaffold; baseline (speedup 1.0000x reference)
import jax
import jax.numpy as jnp
from jax.experimental import pallas as pl


def kernel(x, edge_index, W_in, b_in, g_in, be_in, tm_W, tm_b, tmn_g, tmn_b, W_out, b_out):
    raise NotImplementedError("write your pallas kernel here")


# speedup vs baseline: 6.6133x; 6.6133x over previous
import functools

import jax
import jax.numpy as jnp
from jax import lax
from jax.experimental import pallas as pl
from jax.experimental.pallas import tpu as pltpu
from jax.experimental.pallas import tpu_sc as plsc

N = 10000
E = 320000
HID = 128
OUT_C = 64
CH = 64
NL = 8

NC = 2
NS = 16
NW = NC * NS
EPT = E // NW
B = 80
NCHUNK = EPT // B
FR = 80
NF = N // FR
KMAX = -(-NF // NS)

RB = 2000
GR = N // RB


def _sc_msum_body(x_hbm, src_hbm, dst_hbm, out_hbm, src_v, dst_v, rows_v,
                  acc_s):
    cid = lax.axis_index("c")
    sid = lax.axis_index("s")
    wid = sid * NC + cid
    pltpu.sync_copy(src_hbm.at[wid], src_v)
    pltpu.sync_copy(dst_hbm.at[wid], dst_v)

    def _zero(i, carry):
        rows_v[i // 8, pl.ds((i % 8) * 16, 16)] = jnp.zeros((16,), jnp.float32)
        return carry

    lax.fori_loop(0, FR * 8, _zero, 0)
    for k in range(KMAX):
        c = sid + NS * k

        @pl.when(c < NF)
        def _():
            pltpu.sync_copy(rows_v, acc_s.at[pl.ds(c * FR, FR)])

    plsc.subcore_barrier()

    def _chunk(c, carry):
        pltpu.sync_copy(x_hbm.at[src_v.at[c]], rows_v)
        pltpu.sync_copy(rows_v, acc_s.at[dst_v.at[c]], add=True)
        return carry

    lax.fori_loop(0, NCHUNK, _chunk, 0)
    plsc.subcore_barrier()
    for k in range(KMAX):
        c = sid + NS * k

        @pl.when(c < NF)
        def _():
            pltpu.sync_copy(acc_s.at[pl.ds(c * FR, FR)],
                            out_hbm.at[cid, pl.ds(c * FR, FR)])


def _sc_cnt_body(dst_hbm, out_hbm, dst_v, ones_v, stage_v, acc_s):
    cid = lax.axis_index("c")
    sid = lax.axis_index("s")
    wid = sid * NC + cid
    pltpu.sync_copy(dst_hbm.at[wid], dst_v)

    def _fill(i, carry):
        ones_v[i // 8, pl.ds((i % 8) * 16, 16)] = jnp.ones((16,), jnp.float32)
        return carry

    lax.fori_loop(0, B * 8, _fill, 0)

    def _zero(i, carry):
        stage_v[i // 8, pl.ds((i % 8) * 16, 16)] = jnp.zeros((16,), jnp.float32)
        return carry

    lax.fori_loop(0, FR * 8, _zero, 0)
    for k in range(KMAX):
        c = sid + NS * k

        @pl.when(c < NF)
        def _():
            pltpu.sync_copy(stage_v, acc_s.at[pl.ds(c * FR, FR)])

    plsc.subcore_barrier()

    def _chunk(c, carry):
        pltpu.sync_copy(ones_v, acc_s.at[dst_v.at[c]], add=True)
        return carry

    lax.fori_loop(0, NCHUNK, _chunk, 0)
    plsc.subcore_barrier()
    for k in range(KMAX):
        c = sid + NS * k

        @pl.when(c < NF)
        def _():
            pltpu.sync_copy(acc_s.at[pl.ds(c * FR, FR)],
                            out_hbm.at[cid, pl.ds(c * FR, FR)])


@functools.lru_cache(maxsize=None)
def _build_sc():
    mesh = plsc.VectorSubcoreMesh(core_axis_name="c", subcore_axis_name="s")
    msum = pl.kernel(
        _sc_msum_body,
        out_type=jax.ShapeDtypeStruct((NC, N, HID), jnp.float32),
        mesh=mesh,
        scratch_types=[
            pltpu.VMEM((NCHUNK, B), jnp.int32),
            pltpu.VMEM((NCHUNK, B), jnp.int32),
            pltpu.VMEM((B, HID), jnp.float32),
            pltpu.VMEM_SHARED((N, HID), jnp.float32),
        ],
    )
    cnt = pl.kernel(
        _sc_cnt_body,
        out_type=jax.ShapeDtypeStruct((NC, N, HID), jnp.float32),
        mesh=mesh,
        scratch_types=[
            pltpu.VMEM((NCHUNK, B), jnp.int32),
            pltpu.VMEM((B, HID), jnp.float32),
            pltpu.VMEM((FR, HID), jnp.float32),
            pltpu.VMEM_SHARED((N, HID), jnp.float32),
        ],
    )
    return msum, cnt


def _ln(y, g, b):
    mu = jnp.mean(y, axis=-1, keepdims=True)
    var = jnp.mean((y - mu) * (y - mu), axis=-1, keepdims=True)
    return (y - mu) * lax.rsqrt(var + 1e-5) * g + b


def _mlp_body(x_ref, w0_ref, b0_ref, g0_ref, e0_ref, w1_ref, b1_ref, g1_ref,
              e1_ref, o_ref):
    h = jnp.dot(x_ref[...], w0_ref[...], preferred_element_type=jnp.float32)
    h = jnp.maximum(h + b0_ref[...], 0.0)
    h = _ln(h, g0_ref[...], e0_ref[...])
    h = jnp.dot(h, w1_ref[...], preferred_element_type=jnp.float32)
    h = jnp.maximum(h + b1_ref[...], 0.0)
    o_ref[...] = _ln(h, g1_ref[...], e1_ref[...])


_row_spec = pl.BlockSpec((RB, HID), lambda i: (i, 0))
_full = lambda shape: pl.BlockSpec(shape, lambda i: tuple(0 for _ in shape))

_tc_mlp = pl.pallas_call(
    _mlp_body,
    grid=(GR,),
    in_specs=[
        _row_spec,
        _full((HID, HID)), _full((1, HID)), _full((1, HID)), _full((1, HID)),
        _full((HID, HID)), _full((1, HID)), _full((1, HID)), _full((1, HID)),
    ],
    out_specs=_row_spec,
    out_shape=jax.ShapeDtypeStruct((N, HID), jnp.float32),
)


def _layer_body(final, x_ref, p_ref, cp_ref, ts_ref, wx_ref, wm_ref, b_ref,
                t_ref, r_ref, g_ref, e_ref, wo_ref, bo_ref, xo_ref, tso_ref):
    cnt = cp_ref[0, :, 0:1] + cp_ref[1, :, 0:1]
    cinv = 1.0 / jnp.maximum(cnt, 1.0)
    m = (p_ref[0] + p_ref[1]) * cinv
    x = x_ref[...]
    logits = (jnp.dot(x, wx_ref[...], preferred_element_type=jnp.float32)
              + jnp.dot(m, wm_ref[...], preferred_element_type=jnp.float32)
              + b_ref[...])
    z = logits - jnp.max(logits, axis=-1, keepdims=True)
    ez = jnp.exp(z)
    sm = ez / jnp.sum(ez, axis=-1, keepdims=True)
    cs = jnp.dot(sm, t_ref[...], preferred_element_type=jnp.float32)
    ts = ts_ref[...]
    raw = ts + (1.0 - ts) * cs
    sig = jnp.dot(raw, r_ref[...], preferred_element_type=jnp.float32)
    y = x * sig + m * (1.0 - sig)
    yn = _ln(y, g_ref[...], e_ref[...])
    if final:
        xo_ref[...] = (jnp.dot(yn, wo_ref[...],
                               preferred_element_type=jnp.float32)
                       + bo_ref[...])
    else:
        xo_ref[...] = yn
    tso_ref[...] = raw


def _make_layer(final):
    return pl.pallas_call(
        functools.partial(_layer_body, final),
        grid=(GR,),
        in_specs=[
            _row_spec,
            pl.BlockSpec((NC, RB, HID), lambda i: (0, i, 0)),
            pl.BlockSpec((NC, RB, 16), lambda i: (0, i, 0)),
            pl.BlockSpec((RB, CH), lambda i: (i, 0)),
            _full((HID, CH)), _full((HID, CH)), _full((1, CH)),
            _full((CH, CH)), _full((CH, HID)),
            _full((1, HID)), _full((1, HID)),
            _full((HID, OUT_C)), _full((1, OUT_C)),
        ],
        out_specs=[
            pl.BlockSpec((RB, OUT_C if final else HID), lambda i: (i, 0)),
            pl.BlockSpec((RB, CH), lambda i: (i, 0)),
        ],
        out_shape=[
            jax.ShapeDtypeStruct((N, OUT_C if final else HID), jnp.float32),
            jax.ShapeDtypeStruct((N, CH), jnp.float32),
        ],
    )


_tc_layer = _make_layer(False)
_tc_layer_final = _make_layer(True)


def kernel(x, edge_index, W_in, b_in, g_in, be_in, tm_W, tm_b, tmn_g, tmn_b,
           W_out, b_out):
    _sc_msum, _sc_cnt = _build_sc()
    src = edge_index[0].astype(jnp.int32).reshape(NW, NCHUNK, B)
    dst = edge_index[1].astype(jnp.int32).reshape(NW, NCHUNK, B)
    cp = _sc_cnt(dst)[:, :, :16]
    h = _tc_mlp(x, W_in[0], b_in[0:1], g_in[0:1], be_in[0:1],
                W_in[1], b_in[1:2], g_in[1:2], be_in[1:2])
    tri = jnp.triu(jnp.ones((CH, CH), jnp.float32))
    rep = jnp.repeat(jnp.eye(CH, dtype=jnp.float32), HID // CH, axis=1)
    ts = jnp.zeros((N, CH), jnp.float32)
    for j in range(NL):
        p = _sc_msum(h, src, dst)
        call = _tc_layer_final if j == NL - 1 else _tc_layer
        h, ts = call(h, p, cp, ts, tm_W[j, :HID], tm_W[j, HID:],
                     tm_b[j:j + 1], tri, rep, tmn_g[j:j + 1], tmn_b[j:j + 1],
                     W_out, b_out.reshape(1, OUT_C))
    return h


# speedup vs baseline: 10.3113x; 1.5592x over previous
import functools

import jax
import jax.numpy as jnp
from jax import lax
from jax.experimental import pallas as pl
from jax.experimental.pallas import tpu as pltpu
from jax.experimental.pallas import tpu_sc as plsc

N = 10000
E = 320000
HID = 128
OUT_C = 64
CH = 64
NL = 8

NC = 2
NS = 16
NW = NC * NS
EPT = E // NW
B = 80
NCHUNK = EPT // B
IDXR = 64
HALVES = ((0, 64), (64, 61))
NZ = N // B
KZ = -(-NZ // NS)
FF = 200
NF = N // FF
KF = -(-NF // NS)

RB = 2000
GR = N // RB


def _sc_msum_body(x_hbm, src_hbm, dst_hbm, out_hbm, src_v, dst_v, rows_v,
                  acc_s, gsem0, gsem1):
    cid = lax.axis_index("c")
    sid = lax.axis_index("s")
    wid = sid * NC + cid

    def _zero(i, carry):
        rows_v[0, i // 8, pl.ds((i % 8) * 16, 16)] = jnp.zeros((16,),
                                                               jnp.float32)
        return carry

    lax.fori_loop(0, B * 8, _zero, 0)
    for k in range(KZ):
        c = sid + NS * k

        @pl.when(c < NZ)
        def _():
            pltpu.sync_copy(rows_v.at[0], acc_s.at[pl.ds(c * B, B)])

    plsc.subcore_barrier()

    sems = (gsem0, gsem1)
    for hbase, hn in HALVES:
        pltpu.sync_copy(src_hbm.at[wid, pl.ds(hbase, hn)],
                        src_v.at[pl.ds(0, hn)])
        pltpu.sync_copy(dst_hbm.at[wid, pl.ds(hbase, hn)],
                        dst_v.at[pl.ds(0, hn)])
        pltpu.async_copy(x_hbm.at[src_v.at[0]], rows_v.at[0], gsem0)
        pltpu.async_copy(x_hbm.at[src_v.at[1]], rows_v.at[1], gsem1)

        def _pair(g, carry):
            for u in range(2):
                c = 2 * g + u
                buf = rows_v.at[u]
                pltpu.make_async_copy(x_hbm.at[src_v.at[c]], buf,
                                      sems[u]).wait()
                pltpu.sync_copy(buf, acc_s.at[dst_v.at[c]], add=True)

                @pl.when(c + 2 < hn)
                def _():
                    pltpu.async_copy(x_hbm.at[src_v.at[c + 2]], buf, sems[u])
            return carry

        lax.fori_loop(0, hn // 2, _pair, 0)
        if hn % 2:
            c = hn - 1
            buf = rows_v.at[c % 2]
            pltpu.make_async_copy(x_hbm.at[src_v.at[c]], buf,
                                  sems[c % 2]).wait()
            pltpu.sync_copy(buf, acc_s.at[dst_v.at[c]], add=True)
    plsc.subcore_barrier()
    for k in range(KF):
        c = sid + NS * k

        @pl.when(c < NF)
        def _():
            pltpu.sync_copy(acc_s.at[pl.ds(c * FF, FF)],
                            out_hbm.at[cid, pl.ds(c * FF, FF)])


def _sc_cnt_body(dst_hbm, out_hbm, dst_v, ones_v, stage_v, acc_s):
    cid = lax.axis_index("c")
    sid = lax.axis_index("s")
    wid = sid * NC + cid
    pltpu.sync_copy(dst_hbm.at[wid], dst_v)

    def _fill(i, carry):
        ones_v[i // 8, pl.ds((i % 8) * 16, 16)] = jnp.ones((16,), jnp.float32)
        return carry

    lax.fori_loop(0, B * 8, _fill, 0)

    def _zero(i, carry):
        stage_v[i // 8, pl.ds((i % 8) * 16, 16)] = jnp.zeros((16,), jnp.float32)
        return carry

    lax.fori_loop(0, B * 8, _zero, 0)
    for k in range(KZ):
        c = sid + NS * k

        @pl.when(c < NZ)
        def _():
            pltpu.sync_copy(stage_v, acc_s.at[pl.ds(c * B, B)])

    plsc.subcore_barrier()

    def _chunk(c, carry):
        pltpu.sync_copy(ones_v, acc_s.at[dst_v.at[c]], add=True)
        return carry

    lax.fori_loop(0, NCHUNK, _chunk, 0)
    plsc.subcore_barrier()
    for k in range(KF):
        c = sid + NS * k

        @pl.when(c < NF)
        def _():
            pltpu.sync_copy(acc_s.at[pl.ds(c * FF, FF)],
                            out_hbm.at[cid, pl.ds(c * FF, FF)])


@functools.lru_cache(maxsize=None)
def _build_sc():
    mesh = plsc.VectorSubcoreMesh(core_axis_name="c", subcore_axis_name="s")
    msum = pl.kernel(
        _sc_msum_body,
        out_type=jax.ShapeDtypeStruct((NC, N, HID), jnp.float32),
        mesh=mesh,
        scratch_types=[
            pltpu.VMEM((IDXR, B), jnp.int32),
            pltpu.VMEM((IDXR, B), jnp.int32),
            pltpu.VMEM((2, B, HID), jnp.float32),
            pltpu.VMEM_SHARED((N, HID), jnp.float32),
            pltpu.SemaphoreType.DMA,
            pltpu.SemaphoreType.DMA,
        ],
    )
    cnt = pl.kernel(
        _sc_cnt_body,
        out_type=jax.ShapeDtypeStruct((NC, N, HID), jnp.float32),
        mesh=mesh,
        scratch_types=[
            pltpu.VMEM((NCHUNK, B), jnp.int32),
            pltpu.VMEM((B, HID), jnp.float32),
            pltpu.VMEM((B, HID), jnp.float32),
            pltpu.VMEM_SHARED((N, HID), jnp.float32),
        ],
    )
    return msum, cnt


def _ln(y, g, b):
    mu = jnp.mean(y, axis=-1, keepdims=True)
    var = jnp.mean((y - mu) * (y - mu), axis=-1, keepdims=True)
    return (y - mu) * lax.rsqrt(var + 1e-5) * g + b


def _mlp_body(x_ref, w0_ref, b0_ref, g0_ref, e0_ref, w1_ref, b1_ref, g1_ref,
              e1_ref, o_ref):
    h = jnp.dot(x_ref[...], w0_ref[...], preferred_element_type=jnp.float32)
    h = jnp.maximum(h + b0_ref[...], 0.0)
    h = _ln(h, g0_ref[...], e0_ref[...])
    h = jnp.dot(h, w1_ref[...], preferred_element_type=jnp.float32)
    h = jnp.maximum(h + b1_ref[...], 0.0)
    o_ref[...] = _ln(h, g1_ref[...], e1_ref[...])


_row_spec = pl.BlockSpec((RB, HID), lambda i: (i, 0))
_full = lambda shape: pl.BlockSpec(shape, lambda i: tuple(0 for _ in shape))

_tc_mlp = pl.pallas_call(
    _mlp_body,
    grid=(GR,),
    in_specs=[
        _row_spec,
        _full((HID, HID)), _full((1, HID)), _full((1, HID)), _full((1, HID)),
        _full((HID, HID)), _full((1, HID)), _full((1, HID)), _full((1, HID)),
    ],
    out_specs=_row_spec,
    out_shape=jax.ShapeDtypeStruct((N, HID), jnp.float32),
)


def _layer_body(final, x_ref, p_ref, cp_ref, ts_ref, wx_ref, wm_ref, b_ref,
                t_ref, r_ref, g_ref, e_ref, wo_ref, bo_ref, xo_ref, tso_ref):
    cnt = cp_ref[0, :, 0:1] + cp_ref[1, :, 0:1]
    cinv = 1.0 / jnp.maximum(cnt, 1.0)
    m = (p_ref[0] + p_ref[1]) * cinv
    x = x_ref[...]
    logits = (jnp.dot(x, wx_ref[...], preferred_element_type=jnp.float32)
              + jnp.dot(m, wm_ref[...], preferred_element_type=jnp.float32)
              + b_ref[...])
    z = logits - jnp.max(logits, axis=-1, keepdims=True)
    ez = jnp.exp(z)
    sm = ez / jnp.sum(ez, axis=-1, keepdims=True)
    cs = jnp.dot(sm, t_ref[...], preferred_element_type=jnp.float32)
    ts = ts_ref[...]
    raw = ts + (1.0 - ts) * cs
    sig = jnp.dot(raw, r_ref[...], preferred_element_type=jnp.float32)
    y = x * sig + m * (1.0 - sig)
    yn = _ln(y, g_ref[...], e_ref[...])
    if final:
        xo_ref[...] = (jnp.dot(yn, wo_ref[...],
                               preferred_element_type=jnp.float32)
                       + bo_ref[...])
    else:
        xo_ref[...] = yn
    tso_ref[...] = raw


def _make_layer(final):
    return pl.pallas_call(
        functools.partial(_layer_body, final),
        grid=(GR,),
        in_specs=[
            _row_spec,
            pl.BlockSpec((NC, RB, HID), lambda i: (0, i, 0)),
            pl.BlockSpec((NC, RB, 16), lambda i: (0, i, 0)),
            pl.BlockSpec((RB, CH), lambda i: (i, 0)),
            _full((HID, CH)), _full((HID, CH)), _full((1, CH)),
            _full((CH, CH)), _full((CH, HID)),
            _full((1, HID)), _full((1, HID)),
            _full((HID, OUT_C)), _full((1, OUT_C)),
        ],
        out_specs=[
            pl.BlockSpec((RB, OUT_C if final else HID), lambda i: (i, 0)),
            pl.BlockSpec((RB, CH), lambda i: (i, 0)),
        ],
        out_shape=[
            jax.ShapeDtypeStruct((N, OUT_C if final else HID), jnp.float32),
            jax.ShapeDtypeStruct((N, CH), jnp.float32),
        ],
    )


_tc_layer = _make_layer(False)
_tc_layer_final = _make_layer(True)


def kernel(x, edge_index, W_in, b_in, g_in, be_in, tm_W, tm_b, tmn_g, tmn_b,
           W_out, b_out):
    _sc_msum, _sc_cnt = _build_sc()
    src = edge_index[0].astype(jnp.int32).reshape(NW, NCHUNK, B)
    dst = edge_index[1].astype(jnp.int32).reshape(NW, NCHUNK, B)
    cp = _sc_cnt(dst)[:, :, :16]
    h = _tc_mlp(x, W_in[0], b_in[0:1], g_in[0:1], be_in[0:1],
                W_in[1], b_in[1:2], g_in[1:2], be_in[1:2])
    tri = jnp.triu(jnp.ones((CH, CH), jnp.float32))
    rep = jnp.repeat(jnp.eye(CH, dtype=jnp.float32), HID // CH, axis=1)
    ts = jnp.zeros((N, CH), jnp.float32)
    for j in range(NL):
        p = _sc_msum(h, src, dst)
        call = _tc_layer_final if j == NL - 1 else _tc_layer
        h, ts = call(h, p, cp, ts, tm_W[j, :HID], tm_W[j, HID:],
                     tm_b[j:j + 1], tri, rep, tmn_g[j:j + 1], tmn_b[j:j + 1],
                     W_out, b_out.reshape(1, OUT_C))
    return h
